# baseline (speedup 1.0000x reference)
import functools

import jax
import jax.numpy as jnp
from jax import lax
from jax.experimental import pallas as pl
from jax.experimental.pallas import tpu as pltpu
from jax.experimental.pallas import tpu_sc as plsc

NUM_BINS = 8
TB = 3.0
MIN_BIN_WIDTH = 1e-3
MIN_BIN_HEIGHT = 1e-3
MIN_DERIVATIVE = 1e-3

N = 16777216
NW = 32
PER_W = N // NW
CHUNK = 16384
NCH = PER_W // CHUNK
LANES = 16
NROWS = 13
NCELL = 2048
SCALE = (NCELL - 0.5) / (2 * TB)


def _build_tables(params):
    K = NUM_BINS
    w_raw = params[:K]
    h_raw = params[K:2 * K]
    d_raw = params[2 * K:]
    widths = jax.nn.softmax(w_raw, axis=-1)
    widths = MIN_BIN_WIDTH + (1 - MIN_BIN_WIDTH * K) * widths
    heights = jax.nn.softmax(h_raw, axis=-1)
    heights = MIN_BIN_HEIGHT + (1 - MIN_BIN_HEIGHT * K) * heights
    derivs = MIN_DERIVATIVE + jax.nn.softplus(d_raw)
    widths = 2 * TB * widths
    heights = 2 * TB * heights

    cw = jnp.cumsum(widths)
    cw = jnp.concatenate([jnp.full((1,), -TB, dtype=cw.dtype), cw])
    cw = (cw[:-1] + cw[1:]) / 2
    cw = cw.at[0].set(-TB).at[-1].set(TB)
    ch = jnp.cumsum(heights)
    ch = jnp.concatenate([jnp.full((1,), -TB, dtype=ch.dtype), ch])
    ch = (ch[:-1] + ch[1:]) / 2
    ch = ch.at[0].set(-TB).at[-1].set(TB)

    d = derivs[:K]
    d1 = derivs[1:K + 1]
    delta = heights / widths
    winv = 1.0 / widths
    C = d + d1 - 2 * delta
    tabs = [
        winv,
        -cw * winv,
        ch,
        heights * (delta - d),
        heights * d,
        -C,
        C,
        delta,
        delta * delta * C,
        2 * delta * delta * (delta - d),
        delta * delta * d,
    ]
    M = jnp.where(cw[1:7] <= TB, cw[1:7], 10.0).astype(jnp.float32)
    cur = jnp.concatenate([jnp.full((1,), -10.0, jnp.float32), M])
    nxt = jnp.concatenate([M, jnp.full((1,), 10.0, jnp.float32)])
    edges = (jnp.arange(NCELL, dtype=jnp.float32)
             * jnp.float32(1.0 / SCALE)) - jnp.float32(TB)
    grid = jnp.sum(edges[:, None] >= M[None, :], axis=1).astype(jnp.int32)

    packed = jnp.zeros((NROWS, LANES), jnp.float32)
    for k, t in enumerate(tabs):
        packed = packed.at[k, :K].set(t.astype(jnp.float32))
    packed = packed.at[11, :7].set(cur)
    packed = packed.at[12, :7].set(nxt)
    return packed, grid


def _spline_vec(x, tblv, gridv):
    f32 = jnp.float32
    i32 = jnp.int32
    xc = jnp.minimum(jnp.maximum(x, f32(-TB)), f32(TB))

    uf = xc * f32(SCALE) + f32(TB * SCALE)
    u = uf.astype(i32)
    b0 = plsc.load_gather(gridv, [u])
    curv = plsc.load_gather(tblv.at[11], [b0])
    nxtv = plsc.load_gather(tblv.at[12], [b0])
    b = (b0 - (xc < curv).astype(i32) + (xc >= nxtv).astype(i32)
         + (xc >= f32(TB)).astype(i32))

    winv = plsc.load_gather(tblv.at[0], [b])
    ncwi = plsc.load_gather(tblv.at[1], [b])
    chb = plsc.load_gather(tblv.at[2], [b])
    A2 = plsc.load_gather(tblv.at[3], [b])
    B = plsc.load_gather(tblv.at[4], [b])
    nC = plsc.load_gather(tblv.at[5], [b])
    C = plsc.load_gather(tblv.at[6], [b])
    delta = plsc.load_gather(tblv.at[7], [b])
    E2 = plsc.load_gather(tblv.at[8], [b])
    F2 = plsc.load_gather(tblv.at[9], [b])
    G = plsc.load_gather(tblv.at[10], [b])

    th = xc * winv + ncwi
    num = th * (A2 * th + B)
    den = (nC * th + C) * th + delta
    r = f32(1.0) / den
    out_in = num * r + chb
    dnum = (E2 * th + F2) * th + G
    larg = (dnum + f32(1e-8)) * r * r

    iv = lax.bitcast_convert_type(larg, i32)
    e = lax.shift_right_logical(iv, 23) - 127
    m = lax.bitcast_convert_type((iv & 0x7FFFFF) | 0x3F800000, f32)
    t = m - f32(1.0)
    p = f32(0.03044900453867244)
    p = p * t + f32(-0.13158182508876531)
    p = p * t + f32(0.2852726810905729)
    p = p * t + f32(-0.49023072342340746)
    p = p * t + f32(0.9992354838332744)
    p = p * t + f32(9.975032552228137e-06)
    lad_in = e.astype(f32) * f32(0.6931471805599453) + p

    inside = x == xc
    out = jnp.where(inside, out_in, x)
    lad = jnp.where(inside, lad_in, f32(0.0))
    return out, lad


def _sc_body(x_hbm, tbl_hbm, grid_hbm, out_hbm, lad_hbm, tblv, gridv,
             xin, yout, lout, sin, sy, sl):
    wid = lax.axis_index("s") * 2 + lax.axis_index("c")
    base = wid * PER_W
    pltpu.sync_copy(tbl_hbm, tblv)
    pltpu.sync_copy(grid_hbm, gridv)

    def in_copy(c, slot):
        off = pl.multiple_of(base + c * CHUNK, CHUNK)
        return pltpu.make_async_copy(
            x_hbm.at[pl.ds(off, CHUNK)], xin.at[slot], sin.at[slot])

    def y_copy(c, slot):
        off = pl.multiple_of(base + c * CHUNK, CHUNK)
        return pltpu.make_async_copy(
            yout.at[slot], out_hbm.at[pl.ds(off, CHUNK)], sy.at[slot])

    def l_copy(c, slot):
        off = pl.multiple_of(base + c * CHUNK, CHUNK)
        return pltpu.make_async_copy(
            lout.at[slot], lad_hbm.at[pl.ds(off, CHUNK)], sl.at[slot])

    def process(c, slot, prefetch, drain):
        in_copy(c, slot).wait()
        @pl.when(drain)
        def _():
            y_copy(c - 2, slot).wait()
            l_copy(c - 2, slot).wait()

        @plsc.parallel_loop(0, CHUNK, step=LANES, unroll=8)
        def vec_body(o):
            x = xin[slot, pl.ds(o, LANES)]
            out, lad = _spline_vec(x, tblv, gridv)
            yout[slot, pl.ds(o, LANES)] = out
            lout[slot, pl.ds(o, LANES)] = lad

        y_copy(c, slot).start()
        l_copy(c, slot).start()
        if prefetch:
            in_copy(c + 2, slot).start()

    in_copy(0, 0).start()
    in_copy(1, 1).start()

    def chunk_pair(i, carry):
        c0 = i * 2
        process(c0, 0, True, c0 >= 2)
        process(c0 + 1, 1, True, c0 >= 2)
        return carry

    lax.fori_loop(0, NCH // 2 - 1, chunk_pair, 0)
    process(NCH - 2, 0, False, jnp.bool_(True))
    process(NCH - 1, 1, False, jnp.bool_(True))
    y_copy(NCH - 2, 0).wait()
    l_copy(NCH - 2, 0).wait()
    y_copy(NCH - 1, 1).wait()
    l_copy(NCH - 1, 1).wait()


@jax.jit
def kernel(x, params):
    tbl, grid = _build_tables(params)
    mesh = plsc.VectorSubcoreMesh(core_axis_name="c", subcore_axis_name="s")
    f = pl.kernel(
        _sc_body,
        out_type=(
            jax.ShapeDtypeStruct((N,), jnp.float32),
            jax.ShapeDtypeStruct((N,), jnp.float32),
        ),
        mesh=mesh,
        compiler_params=pltpu.CompilerParams(needs_layout_passes=False),
        scratch_types=[
            pltpu.VMEM((NROWS, LANES), jnp.float32),
            pltpu.VMEM((NCELL,), jnp.int32),
            pltpu.VMEM((2, CHUNK), jnp.float32),
            pltpu.VMEM((2, CHUNK), jnp.float32),
            pltpu.VMEM((2, CHUNK), jnp.float32),
            pltpu.SemaphoreType.DMA((2,)),
            pltpu.SemaphoreType.DMA((2,)),
            pltpu.SemaphoreType.DMA((2,)),
        ],
    )
    return f(x, tbl, grid)


# speedup vs baseline: 1.3528x; 1.3528x over previous
import functools

import jax
import jax.numpy as jnp
from jax import lax
from jax.experimental import pallas as pl
from jax.experimental.pallas import tpu as pltpu
from jax.experimental.pallas import tpu_sc as plsc

NUM_BINS = 8
TB = 3.0
MIN_BIN_WIDTH = 1e-3
MIN_BIN_HEIGHT = 1e-3
MIN_DERIVATIVE = 1e-3

N = 16777216
NW = 32
PER_W = N // NW
CHUNK = 16384
NCH = PER_W // CHUNK
LANES = 16
NROWS = 13
NCELL = 2048
SCALE = (NCELL - 0.5) / (2 * TB)


def _build_tables(params):
    K = NUM_BINS
    w_raw = params[:K]
    h_raw = params[K:2 * K]
    d_raw = params[2 * K:]
    widths = jax.nn.softmax(w_raw, axis=-1)
    widths = MIN_BIN_WIDTH + (1 - MIN_BIN_WIDTH * K) * widths
    heights = jax.nn.softmax(h_raw, axis=-1)
    heights = MIN_BIN_HEIGHT + (1 - MIN_BIN_HEIGHT * K) * heights
    derivs = MIN_DERIVATIVE + jax.nn.softplus(d_raw)
    widths = 2 * TB * widths
    heights = 2 * TB * heights

    cw = jnp.cumsum(widths)
    cw = jnp.concatenate([jnp.full((1,), -TB, dtype=cw.dtype), cw])
    cw = (cw[:-1] + cw[1:]) / 2
    cw = cw.at[0].set(-TB).at[-1].set(TB)
    ch = jnp.cumsum(heights)
    ch = jnp.concatenate([jnp.full((1,), -TB, dtype=ch.dtype), ch])
    ch = (ch[:-1] + ch[1:]) / 2
    ch = ch.at[0].set(-TB).at[-1].set(TB)

    d = derivs[:K]
    d1 = derivs[1:K + 1]
    delta = heights / widths
    winv = 1.0 / widths
    C = d + d1 - 2 * delta
    tabs = [
        winv,
        -cw * winv,
        ch,
        heights * (delta - d),
        heights * d,
        -C,
        C,
        delta,
        delta * delta * C,
        2 * delta * delta * (delta - d),
        delta * delta * d,
    ]
    M = jnp.where(cw[1:7] <= TB, cw[1:7], 10.0).astype(jnp.float32)
    cur = jnp.concatenate([jnp.full((1,), -10.0, jnp.float32), M])
    nxt = jnp.concatenate([M, jnp.full((1,), 10.0, jnp.float32)])
    edges = (jnp.arange(NCELL, dtype=jnp.float32)
             * jnp.float32(1.0 / SCALE)) - jnp.float32(TB)
    grid = jnp.sum(edges[:, None] >= M[None, :], axis=1).astype(jnp.int32)

    packed = jnp.zeros((NROWS, LANES), jnp.float32)
    for k, t in enumerate(tabs):
        packed = packed.at[k, :K].set(t.astype(jnp.float32))
    packed = packed.at[11, :7].set(cur)
    packed = packed.at[12, :7].set(nxt)
    return packed, grid


def _spline_vec(x, tblv, gridv):
    f32 = jnp.float32
    i32 = jnp.int32
    xc = jnp.minimum(jnp.maximum(x, f32(-TB)), f32(TB))

    uf = xc * f32(SCALE) + f32(TB * SCALE)
    u = uf.astype(i32)
    b0 = plsc.load_gather(gridv, [u])
    curv = plsc.load_gather(tblv.at[11], [b0])
    nxtv = plsc.load_gather(tblv.at[12], [b0])
    b = (b0 - (xc < curv).astype(i32) + (xc >= nxtv).astype(i32)
         + (xc >= f32(TB)).astype(i32))

    winv = plsc.load_gather(tblv.at[0], [b])
    ncwi = plsc.load_gather(tblv.at[1], [b])
    chb = plsc.load_gather(tblv.at[2], [b])
    A2 = plsc.load_gather(tblv.at[3], [b])
    B = plsc.load_gather(tblv.at[4], [b])
    nC = plsc.load_gather(tblv.at[5], [b])
    C = plsc.load_gather(tblv.at[6], [b])
    delta = plsc.load_gather(tblv.at[7], [b])
    E2 = plsc.load_gather(tblv.at[8], [b])
    F2 = plsc.load_gather(tblv.at[9], [b])
    G = plsc.load_gather(tblv.at[10], [b])

    th = xc * winv + ncwi
    num = th * (A2 * th + B)
    den = (nC * th + C) * th + delta
    r = f32(1.0) / den
    out_in = num * r + chb
    dnum = (E2 * th + F2) * th + G
    larg = (dnum + f32(1e-8)) * r * r

    iv = lax.bitcast_convert_type(larg, i32)
    base = iv.astype(f32) * f32(0.6931471805599453 / (1 << 23)) \
        + f32(-127.0 * 0.6931471805599453)
    m = lax.bitcast_convert_type((iv & 0x7FFFFF) | 0x3F800000, f32)
    t = m - f32(1.0)
    p = f32(0.03044900453867244)
    p = p * t + f32(-0.13158182508876531)
    p = p * t + f32(0.2852726810905729)
    p = p * t + f32(-0.49023072342340746)
    p = p * t + f32(0.3060883032733291)
    p = p * t + f32(9.975032552228137e-06)
    lad_in = base + p

    inside = x == xc
    out = jnp.where(inside, out_in, x)
    lad = jnp.where(inside, lad_in, f32(0.0))
    return out, lad


def _sc_body(x_hbm, tbl_hbm, grid_hbm, out_hbm, lad_hbm, tblv, gridv,
             xin, yout, lout, sin, sy, sl):
    wid = lax.axis_index("s") * 2 + lax.axis_index("c")
    base = wid * PER_W
    pltpu.sync_copy(tbl_hbm, tblv)
    pltpu.sync_copy(grid_hbm, gridv)

    def in_copy(c, slot):
        off = pl.multiple_of(base + c * CHUNK, CHUNK)
        return pltpu.make_async_copy(
            x_hbm.at[pl.ds(off, CHUNK)], xin.at[slot], sin.at[slot])

    def y_copy(c, slot):
        off = pl.multiple_of(base + c * CHUNK, CHUNK)
        return pltpu.make_async_copy(
            yout.at[slot], out_hbm.at[pl.ds(off, CHUNK)], sy.at[slot])

    def l_copy(c, slot):
        off = pl.multiple_of(base + c * CHUNK, CHUNK)
        return pltpu.make_async_copy(
            lout.at[slot], lad_hbm.at[pl.ds(off, CHUNK)], sl.at[slot])

    def process(c, slot, prefetch, drain):
        in_copy(c, slot).wait()
        @pl.when(drain)
        def _():
            y_copy(c - 2, slot).wait()
            l_copy(c - 2, slot).wait()

        @plsc.parallel_loop(0, CHUNK, step=LANES, unroll=4)
        def vec_body(o):
            x = xin[slot, pl.ds(o, LANES)]
            out, lad = _spline_vec(x, tblv, gridv)
            yout[slot, pl.ds(o, LANES)] = out
            lout[slot, pl.ds(o, LANES)] = lad

        y_copy(c, slot).start()
        l_copy(c, slot).start()
        if prefetch:
            in_copy(c + 2, slot).start()

    in_copy(0, 0).start()
    in_copy(1, 1).start()

    def chunk_pair(i, carry):
        c0 = i * 2
        process(c0, 0, True, c0 >= 2)
        process(c0 + 1, 1, True, c0 >= 2)
        return carry

    lax.fori_loop(0, NCH // 2 - 1, chunk_pair, 0)
    process(NCH - 2, 0, False, jnp.bool_(True))
    process(NCH - 1, 1, False, jnp.bool_(True))
    y_copy(NCH - 2, 0).wait()
    l_copy(NCH - 2, 0).wait()
    y_copy(NCH - 1, 1).wait()
    l_copy(NCH - 1, 1).wait()


@jax.jit
def kernel(x, params):
    tbl, grid = _build_tables(params)
    mesh = plsc.VectorSubcoreMesh(core_axis_name="c", subcore_axis_name="s")
    f = pl.kernel(
        _sc_body,
        out_type=(
            jax.ShapeDtypeStruct((N,), jnp.float32),
            jax.ShapeDtypeStruct((N,), jnp.float32),
        ),
        mesh=mesh,
        compiler_params=pltpu.CompilerParams(needs_layout_passes=False),
        scratch_types=[
            pltpu.VMEM((NROWS, LANES), jnp.float32),
            pltpu.VMEM((NCELL,), jnp.int32),
            pltpu.VMEM((2, CHUNK), jnp.float32),
            pltpu.VMEM((2, CHUNK), jnp.float32),
            pltpu.VMEM((2, CHUNK), jnp.float32),
            pltpu.SemaphoreType.DMA((2,)),
            pltpu.SemaphoreType.DMA((2,)),
            pltpu.SemaphoreType.DMA((2,)),
        ],
    )
    return f(x, tbl, grid)
